# Initial kernel scaffold; baseline (speedup 1.0000x reference)
import jax
import jax.numpy as jnp
from jax.experimental import pallas as pl


def kernel(seq_len, bias):
    raise NotImplementedError("write your pallas kernel here")


# speedup vs baseline: 40.7220x; 40.7220x over previous
import functools

import jax
import jax.numpy as jnp
from jax import lax
from jax.experimental import pallas as pl
from jax.experimental.pallas import tpu as pltpu
from jax.experimental.pallas import tpu_sc as plsc

_H = 16
_S = 2048
_T = 2 * _S - 1
_NSHIFT = 8
_WW = 4096
_LANES = 16


def _expand_body(bias_hbm, out_hbm, bias_v, w_v, sem):
    wid = lax.axis_index("s") * 2 + lax.axis_index("c")
    h = wid // 2
    half = wid % 2

    pltpu.sync_copy(bias_hbm, bias_v)

    lanes = lax.iota(jnp.int32, _LANES)
    hvec = jnp.full((_LANES,), 0, jnp.int32) + h
    for r in range(_NSHIFT):
        def build(b, carry, r=r):
            m0 = b * _LANES
            midx = m0 + lanes
            row_idx = jnp.clip(_T + r - midx, 0, _T - 1)
            vals = plsc.load_gather(bias_v, [row_idx, hvec])
            w_v[r, pl.ds(m0, _LANES)] = vals
            return carry
        lax.fori_loop(0, _WW // _LANES, build, 0)

    q0 = half * 128

    def step(t, carry):
        q = q0 + t
        start = pl.multiple_of(2048 - 8 * q, 8)
        dst0 = h * _S + 8 * q
        copies = [
            pltpu.async_copy(
                w_v.at[r, pl.ds(start, _S)], out_hbm.at[dst0 + r], sem)
            for r in range(_NSHIFT)
        ]
        for c in copies:
            c.wait()
        return carry

    lax.fori_loop(0, 128, step, 0)


@functools.partial(
    pl.kernel,
    out_type=jax.ShapeDtypeStruct((_H * _S, _S), jnp.float32),
    mesh=plsc.VectorSubcoreMesh(core_axis_name="c", subcore_axis_name="s"),
    compiler_params=pltpu.CompilerParams(
        use_tc_tiling_on_sc=False, needs_layout_passes=False),
    scratch_types=[
        pltpu.VMEM((_T, _H), jnp.float32),
        pltpu.VMEM((_NSHIFT, _WW), jnp.float32),
        pltpu.SemaphoreType.DMA,
    ],
)
def _expand(bias_hbm, out_hbm, bias_v, w_v, sem):
    _expand_body(bias_hbm, out_hbm, bias_v, w_v, sem)


def kernel(seq_len, bias):
    del seq_len
    out = _expand(bias)
    return out.reshape(_H, _S, _S)


# speedup vs baseline: 40.8707x; 1.0037x over previous
import functools

import jax
import jax.numpy as jnp
from jax import lax
from jax.experimental import pallas as pl
from jax.experimental.pallas import tpu as pltpu
from jax.experimental.pallas import tpu_sc as plsc

_H = 16
_S = 2048
_T = 2 * _S - 1
_NSHIFT = 8
_WW = 4096
_LANES = 16
_AHEAD = 4


def _expand_body(bias_hbm, out_hbm, bias_v, w_v, sem):
    wid = lax.axis_index("s") * 2 + lax.axis_index("c")
    h = wid // 2
    half = wid % 2

    pltpu.sync_copy(bias_hbm, bias_v)

    lanes = lax.iota(jnp.int32, _LANES)
    hvec = jnp.full((_LANES,), 0, jnp.int32) + h
    for r in range(_NSHIFT):
        def build(b, carry, r=r):
            for u in range(4):
                m0 = b * 4 * _LANES + u * _LANES
                midx = m0 + lanes
                row_idx = jnp.clip(_T + r - midx, 0, _T - 1)
                vals = plsc.load_gather(bias_v, [row_idx, hvec])
                w_v[r, pl.ds(m0, _LANES)] = vals
            return carry
        lax.fori_loop(0, _WW // (4 * _LANES), build, 0)

    q0 = half * 128

    def issue(q):
        start = pl.multiple_of(2048 - 8 * q, 8)
        dst0 = h * _S + 8 * q
        return pltpu.async_copy(
            w_v.at[:, pl.ds(start, _S)], out_hbm.at[pl.ds(dst0, _NSHIFT)],
            sem)

    for t in range(_AHEAD):
        issue(q0 + t)

    def step(t, carry):
        issue(q0 + _AHEAD + t).wait()
        return carry

    lax.fori_loop(0, 128 - _AHEAD, step, 0)

    for _ in range(_AHEAD):
        pltpu.make_async_copy(
            w_v.at[:, pl.ds(0, _S)], out_hbm.at[pl.ds(h * _S, _NSHIFT)],
            sem).wait()


@functools.partial(
    pl.kernel,
    out_type=jax.ShapeDtypeStruct((_H * _S, _S), jnp.float32),
    mesh=plsc.VectorSubcoreMesh(core_axis_name="c", subcore_axis_name="s"),
    compiler_params=pltpu.CompilerParams(
        use_tc_tiling_on_sc=False, needs_layout_passes=False),
    scratch_types=[
        pltpu.VMEM((_T, _H), jnp.float32),
        pltpu.VMEM((_NSHIFT, _WW), jnp.float32),
        pltpu.SemaphoreType.DMA,
    ],
)
def _expand(bias_hbm, out_hbm, bias_v, w_v, sem):
    _expand_body(bias_hbm, out_hbm, bias_v, w_v, sem)


def kernel(seq_len, bias):
    del seq_len
    out = _expand(bias)
    return out.reshape(_H, _S, _S)
